# Initial kernel scaffold; baseline (speedup 1.0000x reference)
import jax
import jax.numpy as jnp
from jax.experimental import pallas as pl


def kernel(x, pos, edge_index, edge_attr, params):
    raise NotImplementedError("write your pallas kernel here")


# speedup vs baseline: 1.5970x; 1.5970x over previous
import functools

import jax
import jax.numpy as jnp
from jax import lax
from jax.experimental import pallas as pl
from jax.experimental.pallas import tpu as pltpu
from jax.experimental.pallas import tpu_sc as plsc

_H = 128
_NN = 10000
_NE = 320000
_TD = 16
_EPS = 1e-5

_NW = 32
_CH = 80
_EPW = _NE // _NW
_NCH = _EPW // _CH
_EPT = _NE // 16
_NCHS = _EPT // _CH
_HW = _H // 2
_RH = 5120
_RHA = _RH + 8
_RPH = _RH // 16

_BE = 3200
_GE = _NE // _BE


def _relu(x):
    return jnp.maximum(x, 0.0)


def _ln(m, g, be):
    mu = jnp.mean(m, axis=-1, keepdims=True)
    var = jnp.mean((m - mu) * (m - mu), axis=-1, keepdims=True)
    return (m - mu) * lax.rsqrt(var + _EPS) * g + be


def _node_in_body(x_ref, pos_ref, E1_ref, P1_ref, b1_ref, W2_ref, b2_ref,
                  W3_ref, b3_ref, g_ref, be_ref, Wa_ref, ba_ref, Wb_ref,
                  node_ref, A_ref, B_ref):
    oh = (x_ref[...] == lax.broadcasted_iota(jnp.int32, (_NN, _TD), 1)
          ).astype(jnp.float32)
    h = jnp.dot(oh, E1_ref[...], preferred_element_type=jnp.float32)
    h = h + jnp.dot(pos_ref[...], P1_ref[...],
                    preferred_element_type=jnp.float32) + b1_ref[...]
    h = _relu(h)
    h = _relu(jnp.dot(h, W2_ref[...], preferred_element_type=jnp.float32)
              + b2_ref[...])
    h = jnp.dot(h, W3_ref[...], preferred_element_type=jnp.float32) + b3_ref[...]
    node = _ln(h, g_ref[...], be_ref[...])
    node_ref[...] = node
    A_ref[...] = jnp.dot(node, Wa_ref[...],
                         preferred_element_type=jnp.float32) + ba_ref[...]
    B_ref[...] = jnp.dot(node, Wb_ref[...], preferred_element_type=jnp.float32)


def _edge_in_body(ea_ref, W1_ref, b1_ref, W2_ref, b2_ref, W3_ref, b3_ref,
                  g_ref, be_ref, edge_ref):
    h = _relu(jnp.dot(ea_ref[...], W1_ref[...],
                      preferred_element_type=jnp.float32) + b1_ref[...])
    h = _relu(jnp.dot(h, W2_ref[...], preferred_element_type=jnp.float32)
              + b2_ref[...])
    h = jnp.dot(h, W3_ref[...], preferred_element_type=jnp.float32) + b3_ref[...]
    edge_ref[...] = _ln(h, g_ref[...], be_ref[...])


def _edge_mp_body(gA_ref, gB_ref, edge_ref, W1c_ref, W2_ref, b2_ref,
                  W3_ref, b3_ref, g_ref, be_ref, msg_ref, eout_ref):
    e = edge_ref[...]
    t = gA_ref[...] + gB_ref[...] + jnp.dot(
        e, W1c_ref[...], preferred_element_type=jnp.float32)
    h = _relu(t)
    h = _relu(jnp.dot(h, W2_ref[...], preferred_element_type=jnp.float32)
              + b2_ref[...])
    m = jnp.dot(h, W3_ref[...], preferred_element_type=jnp.float32) + b3_ref[...]
    msg = _ln(m, g_ref[...], be_ref[...])
    msg_ref[...] = jnp.stack([msg[:, :_HW], msg[:, _HW:]])
    eout_ref[...] = e + msg


def _node_mp_body(node_ref, aggr_ref, Wna_ref, Wnb_ref, bn1_ref, Wn2_ref,
                  bn2_ref, Wn3_ref, bn3_ref, g_ref, be_ref, Wa_ref, ba_ref,
                  Wb_ref, node_out_ref, A_ref, B_ref):
    node = node_ref[...]
    aggr = aggr_ref[...]
    h = jnp.dot(node, Wna_ref[...], preferred_element_type=jnp.float32)
    h = h + jnp.dot(aggr, Wnb_ref[...],
                    preferred_element_type=jnp.float32) + bn1_ref[...]
    h = _relu(h)
    h = _relu(jnp.dot(h, Wn2_ref[...], preferred_element_type=jnp.float32)
              + bn2_ref[...])
    h = jnp.dot(h, Wn3_ref[...], preferred_element_type=jnp.float32) + bn3_ref[...]
    node = node + _ln(h, g_ref[...], be_ref[...])
    node_out_ref[...] = node
    A_ref[...] = jnp.dot(node, Wa_ref[...],
                         preferred_element_type=jnp.float32) + ba_ref[...]
    B_ref[...] = jnp.dot(node, Wb_ref[...], preferred_element_type=jnp.float32)


def _node_out_body(node_ref, V1_ref, c1_ref, V2_ref, c2_ref, V3_ref, c3_ref,
                   out_ref):
    h = _relu(jnp.dot(node_ref[...], V1_ref[...],
                      preferred_element_type=jnp.float32) + c1_ref[...])
    h = _relu(jnp.dot(h, V2_ref[...], preferred_element_type=jnp.float32)
              + c2_ref[...])
    out_ref[...] = jnp.dot(h, V3_ref[...],
                           preferred_element_type=jnp.float32) + c3_ref[...]


@functools.lru_cache(maxsize=None)
def _gather_kernel():
    mesh = plsc.VectorSubcoreMesh(core_axis_name="c", subcore_axis_name="s")

    @functools.partial(
        pl.kernel,
        out_type=(jax.ShapeDtypeStruct((_NE, _H), jnp.float32),
                  jax.ShapeDtypeStruct((_NE, _H), jnp.float32)),
        mesh=mesh,
        scratch_types=[
            pltpu.VMEM((_CH,), jnp.int32),
            pltpu.VMEM((_CH,), jnp.int32),
            pltpu.VMEM((_CH, _H), jnp.float32),
            pltpu.VMEM((_CH, _H), jnp.float32),
            pltpu.SemaphoreType.DMA,
            pltpu.SemaphoreType.DMA,
        ],
    )
    def k(A_h, B_h, dst_h, src_h, gA_h, gB_h, di, si, ra, rb, sa, sb):
        wid = lax.axis_index("s") * 2 + lax.axis_index("c")
        base = wid * _EPW

        def body(i, carry):
            off = base + i * _CH
            pltpu.sync_copy(dst_h.at[pl.ds(off, _CH)], di)
            pltpu.sync_copy(src_h.at[pl.ds(off, _CH)], si)
            ca = pltpu.async_copy(A_h.at[di], ra, sa)
            cb = pltpu.async_copy(B_h.at[si], rb, sb)
            ca.wait()
            cb.wait()
            pltpu.sync_copy(ra, gA_h.at[pl.ds(off, _CH)])
            pltpu.sync_copy(rb, gB_h.at[pl.ds(off, _CH)])
            return carry

        lax.fori_loop(0, _NCH, body, 0)

    return k


@functools.lru_cache(maxsize=None)
def _scatter_kernel(half):
    mesh = plsc.VectorSubcoreMesh(core_axis_name="c", subcore_axis_name="s")
    base_row = half * _RH

    @functools.partial(
        pl.kernel,
        out_type=jax.ShapeDtypeStruct((2, _RH, _HW), jnp.float32),
        mesh=mesh,
        scratch_types=[
            pltpu.VMEM((_CH,), jnp.int32),
            pltpu.VMEM((_CH,), jnp.int32),
            pltpu.VMEM((_CH, _HW), jnp.float32),
            pltpu.VMEM((_RPH, _HW), jnp.float32),
            pltpu.VMEM_SHARED((_RHA, _HW), jnp.float32),
        ],
    )
    def k(msg_h, dst_h, out_h, di, dj, mv, zb, aggr):
        c = lax.axis_index("c")
        s = lax.axis_index("s")
        zero = jnp.zeros((16,), jnp.float32)
        cbase = jnp.full((16,), base_row, jnp.int32)
        cn = jnp.full((16,), _RH, jnp.int32)
        cgar = jnp.full((16,), _RH, jnp.int32)
        c7 = jnp.full((16,), 7, jnp.int32)
        czero = jnp.zeros((16,), jnp.int32)

        def zrow(r, carry):
            for j in range(_HW // 16):
                zb[r, pl.ds(j * 16, 16)] = zero
            return carry

        lax.fori_loop(0, _RPH, zrow, 0)
        pltpu.sync_copy(zb, aggr.at[pl.ds(s * _RPH, _RPH)])
        plsc.subcore_barrier()

        base = s * _EPT

        def body(i, carry):
            off = base + i * _CH
            pltpu.sync_copy(dst_h.at[pl.ds(off, _CH)], di)
            pltpu.sync_copy(msg_h.at[c, pl.ds(off, _CH)], mv)
            for j in range(_CH // 16):
                t = di[pl.ds(16 * j, 16)] - cbase
                oob = jnp.logical_or(t < czero, t >= cn)
                garb = cgar + lax.bitwise_and(t, c7)
                dj[pl.ds(16 * j, 16)] = jnp.where(oob, garb, t)
            pltpu.sync_copy(mv, aggr.at[dj], add=True)
            return carry

        lax.fori_loop(0, _NCHS, body, 0)
        plsc.subcore_barrier()
        pltpu.sync_copy(aggr.at[pl.ds(s * _RPH, _RPH)],
                        out_h.at[c, pl.ds(s * _RPH, _RPH)])

    return k


def _sc_gather(A, B, dst, src):
    return _gather_kernel()(A, B, dst, src)


def _sc_scatter(msg, dst):
    top = _scatter_kernel(0)(msg, dst)
    bot = _scatter_kernel(1)(msg, dst)
    return top, bot


def _decode_aggr(parts):
    top, bot = parts
    t = jnp.concatenate([top[0], top[1]], axis=-1)
    b = jnp.concatenate([bot[0], bot[1]], axis=-1)
    return jnp.concatenate([t, b], axis=0)[:_NN]


def _r(b):
    return b.reshape(1, _H)


def kernel(x, pos, edge_index, edge_attr, params):
    f32 = jnp.float32
    sds = jax.ShapeDtypeStruct

    x2 = x.astype(jnp.int32).reshape(_NN, 1)
    posp = jnp.pad(pos.astype(f32), ((0, 0), (0, _TD - pos.shape[1])))
    src = edge_index[0].astype(jnp.int32)
    dst = edge_index[1].astype(jnp.int32)
    eap = jnp.pad(edge_attr.astype(f32), ((0, 0), (0, 8 - edge_attr.shape[1])))

    p = params

    (W1, b1), (W2, b2), (W3, b3) = p["node_in"]["lin"]
    g_ni, be_ni = p["node_in"]["ln"]
    E1 = jnp.zeros((_TD, _H), f32).at[:p["embed"].shape[0]].set(
        p["embed"] @ W1[:_TD])
    P1 = jnp.zeros((_TD, _H), f32).at[:W1.shape[0] - _TD].set(W1[_TD:])

    mp = []
    for layer in p["mp"]:
        (We1, be1), (We2, be2), (We3, be3) = layer["lin_edge"]["lin"]
        ge, bee = layer["lin_edge"]["ln"]
        (Wn1, bn1), (Wn2, bn2), (Wn3, bn3) = layer["lin_node"]["lin"]
        gn, ben = layer["lin_node"]["ln"]
        mp.append(dict(
            Wa=We1[:_H], Wb=We1[_H:2 * _H], Wc=We1[2 * _H:], be1=be1,
            We2=We2, be2=be2, We3=We3, be3=be3, ge=ge, bee=bee,
            Wna=Wn1[:_H], Wnb=Wn1[_H:], bn1=bn1, Wn2=Wn2, bn2=bn2,
            Wn3=Wn3, bn3=bn3, gn=gn, ben=ben))

    wspec = pl.BlockSpec((_H, _H), lambda i: (0, 0))
    vspec = pl.BlockSpec((1, _H), lambda i: (0, 0))
    espec = pl.BlockSpec((_BE, _H), lambda i: (i, 0))

    node, A, B = pl.pallas_call(
        _node_in_body,
        out_shape=(sds((_NN, _H), f32),) * 3,
    )(x2, posp, E1, P1, _r(b1), W2, _r(b2), W3, _r(b3), _r(g_ni), _r(be_ni),
      mp[0]["Wa"], _r(mp[0]["be1"]), mp[0]["Wb"])

    (We1i, be1i), (We2i, be2i), (We3i, be3i) = p["edge_in"]["lin"]
    g_ei, be_ei = p["edge_in"]["ln"]
    We1ip = jnp.zeros((8, _H), f32).at[:We1i.shape[0]].set(We1i)
    edge = pl.pallas_call(
        _edge_in_body,
        grid=(_GE,),
        in_specs=[pl.BlockSpec((_BE, 8), lambda i: (i, 0)),
                  pl.BlockSpec((8, _H), lambda i: (0, 0)), vspec,
                  wspec, vspec, wspec, vspec, vspec, vspec],
        out_specs=espec,
        out_shape=sds((_NE, _H), f32),
    )(eap, We1ip, _r(be1i), We2i, _r(be2i), We3i, _r(be3i), _r(g_ei), _r(be_ei))

    hspec = pl.BlockSpec((2, _BE, _HW), lambda i: (0, i, 0))
    for i, L in enumerate(mp):
        gA, gB = _sc_gather(A, B, dst, src)
        msg, edge = pl.pallas_call(
            _edge_mp_body,
            grid=(_GE,),
            in_specs=[espec, espec, espec, wspec, wspec, vspec, wspec,
                      vspec, vspec, vspec],
            out_specs=(hspec, espec),
            out_shape=(sds((2, _NE, _HW), f32), sds((_NE, _H), f32)),
        )(gA, gB, edge, L["Wc"], L["We2"], _r(L["be2"]), L["We3"],
          _r(L["be3"]), _r(L["ge"]), _r(L["bee"]))
        aggr = _decode_aggr(_sc_scatter(msg, dst))
        nxt = mp[i + 1] if i + 1 < len(mp) else mp[0]
        node, A, B = pl.pallas_call(
            _node_mp_body,
            out_shape=(sds((_NN, _H), f32),) * 3,
        )(node, aggr, L["Wna"], L["Wnb"], _r(L["bn1"]), L["Wn2"],
          _r(L["bn2"]), L["Wn3"], _r(L["bn3"]), _r(L["gn"]), _r(L["ben"]),
          nxt["Wa"], _r(nxt["be1"]), nxt["Wb"])

    (V1, c1), (V2, c2), (V3, c3) = p["node_out"]["lin"]
    V3p = jnp.zeros((_H, 8), f32).at[:, :V3.shape[1]].set(V3)
    c3p = jnp.zeros((8,), f32).at[:V3.shape[1]].set(c3)
    outp = pl.pallas_call(
        _node_out_body,
        out_shape=sds((_NN, 8), f32),
    )(node, V1, _r(c1), V2, _r(c2), V3p, c3p.reshape(1, 8))
    return outp[:, :V3.shape[1]]
